# Initial kernel scaffold; baseline (speedup 1.0000x reference)
import jax
import jax.numpy as jnp
from jax.experimental import pallas as pl


def kernel(x, edge_index, edge_weight, W_l1, b_l1, W_r1, b_r1, W_l2, b_l2, W_r2, b_r2, ln_gamma, ln_beta):
    raise NotImplementedError("write your pallas kernel here")


# speedup vs baseline: 4.3873x; 4.3873x over previous
import functools

import jax
import jax.numpy as jnp
from jax import lax
from jax.experimental import pallas as pl
from jax.experimental.pallas import tpu as pltpu
from jax.experimental.pallas import tpu_sc as plsc

N = 10000
E = 320000
D = 128

NC = 2
NS = 16
NW = NC * NS
K = 128
CHUNKS = 79
EPW = K * CHUNKS
EREAL = E // NW
NPAD = 10240
RPS = NPAD // NS


def _sc_body(xl, src, dst, ew, zer_d, out_sum,
             src_c, dst_c, ew_c, rows_v, sem, acc):
    cid = lax.axis_index("c")
    sid = lax.axis_index("s")
    wid = cid * NS + sid

    pltpu.sync_copy(zer_d.at[pl.ds(sid * RPS, RPS)],
                    acc.at[pl.ds(sid * RPS, RPS)])
    plsc.subcore_barrier()

    def chunk_body(i, carry):
        pltpu.sync_copy(src.at[wid, i], src_c)
        pltpu.sync_copy(dst.at[wid, i], dst_c)
        pltpu.sync_copy(ew.at[wid, i], ew_c)
        pltpu.async_copy(xl.at[src_c], rows_v, sem).wait()

        def grp_body(g, c2):
            ww = ew_c[pl.ds(g * 16, 16)]
            for l in range(16):
                wv = ww[l]
                r = g * 16 + l
                for j in range(D // 16):
                    rows_v[r, pl.ds(j * 16, 16)] = (
                        rows_v[r, pl.ds(j * 16, 16)] * wv)
            return c2

        lax.fori_loop(0, K // 16, grp_body, 0)

        pltpu.sync_copy(rows_v, acc.at[dst_c], add=True)
        return carry

    lax.fori_loop(0, CHUNKS, chunk_body, 0)

    plsc.subcore_barrier()
    pltpu.sync_copy(acc.at[pl.ds(sid * RPS, RPS)],
                    out_sum.at[cid, pl.ds(sid * RPS, RPS)])


def _make_sc_aggregate():
    return functools.partial(
        pl.kernel,
        out_type=jax.ShapeDtypeStruct((NC, NPAD, D), jnp.float32),
        mesh=plsc.VectorSubcoreMesh(core_axis_name="c", subcore_axis_name="s"),
        scratch_types=[
            pltpu.VMEM((K,), jnp.int32),
            pltpu.VMEM((K,), jnp.int32),
            pltpu.VMEM((K,), jnp.float32),
            pltpu.VMEM((K, D), jnp.float32),
            pltpu.SemaphoreType.DMA,
            pltpu.VMEM_SHARED((NPAD, D), jnp.float32),
        ],
    )(_sc_body)


def _sc_count_body(dst, zer_d, ones_c, out_cnt, dst_c, ones_v, acc):
    cid = lax.axis_index("c")
    sid = lax.axis_index("s")
    wid = cid * NS + sid

    pltpu.sync_copy(zer_d.at[pl.ds(sid * RPS, RPS)],
                    acc.at[pl.ds(sid * RPS, RPS)])
    pltpu.sync_copy(ones_c, ones_v)
    plsc.subcore_barrier()

    def chunk_body(i, carry):
        pltpu.sync_copy(dst.at[wid, i], dst_c)
        pltpu.sync_copy(ones_v, acc.at[dst_c], add=True)
        return carry

    lax.fori_loop(0, CHUNKS, chunk_body, 0)

    plsc.subcore_barrier()
    pltpu.sync_copy(acc.at[pl.ds(sid * RPS, RPS)],
                    out_cnt.at[cid, pl.ds(sid * RPS, RPS)])


def _make_sc_count():
    return functools.partial(
        pl.kernel,
        out_type=jax.ShapeDtypeStruct((NC, NPAD, D), jnp.float32),
        mesh=plsc.VectorSubcoreMesh(core_axis_name="c", subcore_axis_name="s"),
        scratch_types=[
            pltpu.VMEM((K,), jnp.int32),
            pltpu.VMEM((K, D), jnp.float32),
            pltpu.VMEM_SHARED((NPAD, D), jnp.float32),
        ],
    )(_sc_count_body)


BN = 2000


def _tc1_body(x_ref, w_ref, b_ref, o_ref):
    o_ref[...] = lax.dot_general(
        x_ref[...], w_ref[...], (((1,), (1,)), ((), ())),
        preferred_element_type=jnp.float32) + b_ref[...]


def _tc1(x, w, b):
    return pl.pallas_call(
        _tc1_body,
        grid=(N // BN,),
        in_specs=[
            pl.BlockSpec((BN, D), lambda i: (i, 0)),
            pl.BlockSpec((D, D), lambda i: (0, 0)),
            pl.BlockSpec((1, D), lambda i: (0, 0)),
        ],
        out_specs=pl.BlockSpec((BN, D), lambda i: (i, 0)),
        out_shape=jax.ShapeDtypeStruct((N, D), jnp.float32),
    )(x, w, b)


def _tc2_body(s0_ref, s1_ref, c0_ref, c1_ref, w_ref, b_ref, h_ref, xl2_ref):
    cnt = jnp.maximum((c0_ref[...] + c1_ref[...])[:, 0:1], 1.0)
    h = (s0_ref[...] + s1_ref[...]) / cnt
    h_ref[...] = h
    xl2_ref[...] = lax.dot_general(
        h, w_ref[...], (((1,), (1,)), ((), ())),
        preferred_element_type=jnp.float32) + b_ref[...]


def _tc2(s0, s1, c0, c1, w, b):
    return pl.pallas_call(
        _tc2_body,
        grid=(N // BN,),
        in_specs=[
            pl.BlockSpec((BN, D), lambda i: (i, 0)),
            pl.BlockSpec((BN, D), lambda i: (i, 0)),
            pl.BlockSpec((BN, D), lambda i: (i, 0)),
            pl.BlockSpec((BN, D), lambda i: (i, 0)),
            pl.BlockSpec((D, D), lambda i: (0, 0)),
            pl.BlockSpec((1, D), lambda i: (0, 0)),
        ],
        out_specs=(pl.BlockSpec((BN, D), lambda i: (i, 0)),
                   pl.BlockSpec((BN, D), lambda i: (i, 0))),
        out_shape=(jax.ShapeDtypeStruct((N, D), jnp.float32),
                   jax.ShapeDtypeStruct((N, D), jnp.float32)),
    )(s0, s1, c0, c1, w, b)


def _tc3_body(s0_ref, s1_ref, xl2_ref, h_ref, c0_ref, c1_ref, g_ref, b_ref,
              y_ref):
    cnt2 = (c0_ref[...] + c1_ref[...])[:, 0:1] + 1.0
    out = (s0_ref[...] + s1_ref[...] + xl2_ref[...]) / cnt2
    t = out + h_ref[...]
    mu = jnp.mean(t, axis=-1, keepdims=True)
    var = jnp.mean((t - mu) ** 2, axis=-1, keepdims=True)
    ln = (t - mu) * lax.rsqrt(var + 1e-5) * g_ref[...] + b_ref[...]
    y_ref[...] = ln * 0.5 * (1.0 + lax.erf(ln * 0.7071067811865476))


def _tc3(s0, s1, xl2, h, c0, c1, g, b):
    return pl.pallas_call(
        _tc3_body,
        grid=(N // BN,),
        in_specs=[
            pl.BlockSpec((BN, D), lambda i: (i, 0)),
            pl.BlockSpec((BN, D), lambda i: (i, 0)),
            pl.BlockSpec((BN, D), lambda i: (i, 0)),
            pl.BlockSpec((BN, D), lambda i: (i, 0)),
            pl.BlockSpec((BN, D), lambda i: (i, 0)),
            pl.BlockSpec((BN, D), lambda i: (i, 0)),
            pl.BlockSpec((1, D), lambda i: (0, 0)),
            pl.BlockSpec((1, D), lambda i: (0, 0)),
        ],
        out_specs=pl.BlockSpec((BN, D), lambda i: (i, 0)),
        out_shape=jax.ShapeDtypeStruct((N, D), jnp.float32),
    )(s0, s1, xl2, h, c0, c1, g, b)


def kernel(x, edge_index, edge_weight, W_l1, b_l1, W_r1, b_r1,
           W_l2, b_l2, W_r2, b_r2, ln_gamma, ln_beta):
    ei = edge_index.astype(jnp.int32)
    pad = EPW - EREAL
    src3 = jnp.pad(ei[0].reshape(NW, EREAL),
                   ((0, 0), (0, pad))).reshape(NW, CHUNKS, K)
    dst3 = jnp.pad(ei[1].reshape(NW, EREAL), ((0, 0), (0, pad)),
                   constant_values=NPAD - 1).reshape(NW, CHUNKS, K)
    ew3 = jnp.pad(edge_weight.reshape(NW, EREAL),
                  ((0, 0), (0, pad))).reshape(NW, CHUNKS, K)
    zer_d = jnp.zeros((NPAD, D), jnp.float32)
    ones_c = jnp.ones((K, D), jnp.float32)

    sc_aggregate = _make_sc_aggregate()
    sc_count = _make_sc_count()

    cnt = sc_count(dst3, zer_d, ones_c)
    xl1 = _tc1(x, W_l1, b_l1.reshape(1, D))
    sum1 = sc_aggregate(xl1, src3, dst3, ew3, zer_d)
    h, xl2 = _tc2(sum1[0], sum1[1], cnt[0], cnt[1], W_l2, b_l2.reshape(1, D))
    sum2 = sc_aggregate(xl2, src3, dst3, ew3, zer_d)
    return _tc3(sum2[0], sum2[1], xl2, h, cnt[0], cnt[1],
                ln_gamma.reshape(1, D), ln_beta.reshape(1, D))
